# baseline (speedup 1.0000x reference)
import functools

import jax
import jax.numpy as jnp
from jax import lax
from jax.experimental import pallas as pl
from jax.experimental.pallas import tpu as pltpu
from jax.experimental.pallas import tpu_sc as plsc

_INFO = plsc.get_sparse_core_info()
_NC = _INFO.num_cores
_NS = _INFO.num_subcores
_NW = _NC * _NS
_CHUNK = 128
D = 64


def _sc_gather_body(uidx_hbm, midx_hbm, user_t, movie_t,
                    uv_out, mv_out,
                    uidx_v, midx_v, uv_v, mv_v, sem):
    k = uidx_v.shape[0]
    bpw = k * _CHUNK
    wid = lax.axis_index("s") * _NC + lax.axis_index("c")
    base = wid * bpw
    pltpu.sync_copy(uidx_hbm.at[wid], uidx_v)
    pltpu.sync_copy(midx_hbm.at[wid], midx_v)
    cps = []
    for j in range(k):
        sl = pl.ds(j * _CHUNK, _CHUNK)
        cps.append(pltpu.async_copy(user_t.at[uidx_v.at[j]], uv_v.at[sl], sem))
        cps.append(pltpu.async_copy(movie_t.at[midx_v.at[j]], mv_v.at[sl], sem))
    for cp in cps:
        cp.wait()
    out_sl = pl.ds(base, bpw)
    pltpu.sync_copy(uv_v, uv_out.at[out_sl])
    pltpu.sync_copy(mv_v, mv_out.at[out_sl])


def _sc_gather(uidx, midx, user_t, movie_t):
    B = uidx.size
    k = uidx.shape[1]
    bpw = k * _CHUNK
    mesh = plsc.VectorSubcoreMesh(core_axis_name="c", subcore_axis_name="s")
    fn = pl.kernel(
        _sc_gather_body,
        out_type=(
            jax.ShapeDtypeStruct((B, D), jnp.float32),
            jax.ShapeDtypeStruct((B, D), jnp.float32),
        ),
        mesh=mesh,
        scratch_types=[
            pltpu.VMEM((k, _CHUNK), jnp.int32),
            pltpu.VMEM((k, _CHUNK), jnp.int32),
            pltpu.VMEM((bpw, D), jnp.float32),
            pltpu.VMEM((bpw, D), jnp.float32),
            pltpu.SemaphoreType.DMA,
        ],
        compiler_params=pltpu.CompilerParams(use_tc_tiling_on_sc=False),
    )
    return fn(uidx, midx, user_t, movie_t)


def _tc_dense_body(uv_ref, mv_ref, g_ref, a_ref,
                   gt_ref, at_ref, gbt_ref, abt_ref,
                   w1_ref, b1_ref, w2_ref, b2_ref, w3_ref, b3_ref,
                   wo_ref, bo_ref, out_ref):
    uv = uv_ref[...]
    mv = mv_ref[...]
    g = g_ref[...]
    a = a_ref[...]
    gt = gt_ref[...]
    at = at_ref[...]
    gbt = gbt_ref[...]
    abt = abt_ref[...]
    gv = jnp.where(g == 0, gt[0:1, :], gt[1:2, :])
    gb = jnp.where(g == 0, gbt[0:1, :], gbt[1:2, :])
    na = at.shape[0]
    av = (a == 0).astype(jnp.float32) * at[0:1, :]
    ab = (a == 0).astype(jnp.float32) * abt[0:1, :]
    for i in range(1, na):
        sel = (a == i).astype(jnp.float32)
        av = av + sel * at[i:i + 1, :]
        ab = ab + sel * abt[i:i + 1, :]
    ga = gv + av
    dot = jnp.sum(uv * (mv + ga) + mv * ga + gv * av, axis=1, keepdims=True)
    x = dot + gb + ab
    h = jax.nn.relu(x * w1_ref[...] + b1_ref[...])
    h = jax.nn.relu(jnp.dot(h, w2_ref[...],
                            preferred_element_type=jnp.float32) + b2_ref[...])
    h = jax.nn.relu(jnp.dot(h, w3_ref[...],
                            preferred_element_type=jnp.float32) + b3_ref[...])
    o = jnp.dot(h, wo_ref[...], preferred_element_type=jnp.float32) + bo_ref[...]
    out_ref[...] = jax.nn.sigmoid(o)


def _tc_dense(uv, mv, g, a, gt, at, gbt, abt,
              W1, b1, W2, b2, W3, b3, Wo, bo, block):
    B = uv.shape[0]
    grid = (B // block,)

    def row_spec(shape):
        return pl.BlockSpec((block,) + shape[1:], lambda i: (i,) + (0,) * (len(shape) - 1))

    def full_spec(shape):
        return pl.BlockSpec(shape, lambda i: (0,) * len(shape))

    args = (uv, mv, g, a, gt, at, gbt, abt,
            W1, b1, W2, b2, W3, b3, Wo, bo)
    in_specs = [row_spec(uv.shape), row_spec(mv.shape),
                row_spec(g.shape), row_spec(a.shape)]
    in_specs += [full_spec(x.shape) for x in args[4:]]
    return pl.pallas_call(
        _tc_dense_body,
        grid=grid,
        in_specs=in_specs,
        out_specs=row_spec((B, 1)),
        out_shape=jax.ShapeDtypeStruct((B, 1), jnp.float32),
    )(*args)


def kernel(inputs, user_table, user_bias_table, movie_table, movie_bias_table,
           gender_table, gender_bias_table, age_table, age_bias_table,
           W1, b1, W2, b2, W3, b3, Wo, bo):
    B = inputs.shape[0]
    k = B // (_NW * _CHUNK)
    uidx = inputs[:, 0].reshape(_NW, k, _CHUNK)
    midx = inputs[:, 1].reshape(_NW, k, _CHUNK)
    g = inputs[:, 2:3]
    a = inputs[:, 3:4]
    uv, mv = _sc_gather(uidx, midx, user_table, movie_table)
    return _tc_dense(uv, mv, g, a,
                     gender_table, age_table, gender_bias_table, age_bias_table,
                     W1, b1.reshape(1, -1), W2, b2.reshape(1, -1),
                     W3, b3.reshape(1, -1), Wo, bo.reshape(1, -1),
                     block=4096)
